# baseline (speedup 1.0000x reference)
import functools

import jax
import jax.numpy as jnp
from jax import lax
from jax.experimental import pallas as pl
from jax.experimental.pallas import tpu as pltpu
from jax.experimental.pallas import tpu_sc as plsc

NC = 2
NS = 16
L = 16
NW = NC * NS
EMB = 32
NEG = 5
IDX_CHUNK = 128


def _sc_scores(u_emb, v_emb, u, v, neg_flat):
    B = u.shape[0]
    b_per_w = B // NW
    nneg_w = b_per_w * NEG
    mesh = plsc.VectorSubcoreMesh(core_axis_name="c", subcore_axis_name="s")

    @functools.partial(
        pl.kernel,
        mesh=mesh,
        compiler_params=pltpu.CompilerParams(
            needs_layout_passes=False, use_tc_tiling_on_sc=False),
        out_type=[
            jax.ShapeDtypeStruct((B,), jnp.float32),
            jax.ShapeDtypeStruct((B,), jnp.float32),
        ],
        scratch_types=[
            pltpu.VMEM((b_per_w,), jnp.int32),
            pltpu.VMEM((b_per_w,), jnp.int32),
            pltpu.VMEM((nneg_w,), jnp.int32),
            pltpu.VMEM((b_per_w, EMB), jnp.float32),
            pltpu.VMEM((b_per_w, EMB), jnp.float32),
            pltpu.VMEM((nneg_w, EMB), jnp.float32),
            pltpu.VMEM((b_per_w,), jnp.float32),
            pltpu.VMEM((b_per_w,), jnp.float32),
            pltpu.SemaphoreType.DMA,
        ],
    )
    def sc_kernel(u_emb_hbm, v_emb_hbm, u_hbm, v_hbm, neg_hbm,
                  pos_hbm, negs_hbm,
                  uidx, vidx, nidx, urows, vrows, nrows, pos_s, neg_s, sem):
        wid = lax.axis_index("s") * NC + lax.axis_index("c")
        base = wid * b_per_w
        nbase = wid * nneg_w

        pltpu.sync_copy(u_hbm.at[pl.ds(base, b_per_w)], uidx)
        pltpu.sync_copy(v_hbm.at[pl.ds(base, b_per_w)], vidx)
        pltpu.sync_copy(neg_hbm.at[pl.ds(nbase, nneg_w)], nidx)

        copies = []
        for c in range(b_per_w // IDX_CHUNK):
            s = pl.ds(c * IDX_CHUNK, IDX_CHUNK)
            copies.append(pltpu.async_copy(u_emb_hbm.at[uidx.at[s]], urows.at[s], sem))
            copies.append(pltpu.async_copy(v_emb_hbm.at[vidx.at[s]], vrows.at[s], sem))
        for c in range(nneg_w // IDX_CHUNK):
            s = pl.ds(c * IDX_CHUNK, IDX_CHUNK)
            copies.append(pltpu.async_copy(v_emb_hbm.at[nidx.at[s]], nrows.at[s], sem))
        for cp in copies:
            cp.wait()

        lane = lax.iota(jnp.int32, L)
        zero = jnp.zeros((L,), jnp.float32)

        def group(g, carry):
            rows16 = g * L + lane
            nrow0 = rows16 * NEG

            def jbody(j, accs):
                ap, an = accs
                cj = jnp.full((L,), j, dtype=jnp.int32)
                uj = plsc.load_gather(urows, [rows16, cj])
                vj = plsc.load_gather(vrows, [rows16, cj])
                nj = plsc.load_gather(nrows, [nrow0, cj])
                for n in range(1, NEG):
                    nj = nj + plsc.load_gather(nrows, [nrow0 + n, cj])
                return ap + uj * vj, an + uj * nj

            ap, an = lax.fori_loop(0, EMB, jbody, (zero, zero))
            pos_s[pl.ds(g * L, L)] = ap
            neg_s[pl.ds(g * L, L)] = an
            return carry

        lax.fori_loop(0, b_per_w // L, group, 0)

        pltpu.sync_copy(pos_s, pos_hbm.at[pl.ds(base, b_per_w)])
        pltpu.sync_copy(neg_s, negs_hbm.at[pl.ds(base, b_per_w)])

    return sc_kernel(u_emb, v_emb, u, v, neg_flat)


def _tc_loss(pos, neg):
    B = pos.size
    p2 = pos.reshape(128, B // 128)
    n2 = neg.reshape(128, B // 128)

    def body(p_ref, n_ref, o_ref):
        p = jnp.clip(p_ref[...], -10.0, 10.0)
        n = jnp.clip(n_ref[...], -10.0, 10.0)
        val = jnp.log1p(jnp.exp(-p)) + jnp.log1p(jnp.exp(n))
        o_ref[...] = jnp.full((1, 1), jnp.sum(val) / B, jnp.float32)

    return pl.pallas_call(
        body,
        out_shape=jax.ShapeDtypeStruct((1, 1), jnp.float32),
    )(p2, n2)


def kernel(u_emb, v_emb, u, v, neg_v):
    pos, negs = _sc_scores(
        u_emb, v_emb,
        u.astype(jnp.int32), v.astype(jnp.int32),
        neg_v.reshape(-1).astype(jnp.int32),
    )
    return _tc_loss(pos, negs)[0, 0]
